# Initial kernel scaffold; baseline (speedup 1.0000x reference)
import jax
import jax.numpy as jnp
from jax.experimental import pallas as pl


def kernel(embeds, codebooks):
    raise NotImplementedError("write your pallas kernel here")


# speedup vs baseline: 3.0426x; 3.0426x over previous
import functools

import jax
import jax.numpy as jnp
from jax import lax
from jax.experimental import pallas as pl
from jax.experimental.pallas import tpu as pltpu
from jax.experimental.pallas import tpu_sc as plsc

H = 8
K = 1024
D = 256
N = 8192
BETA = 0.25

BN = 512
NB = N // BN

NC = 2
NS = 16
NW = NC * NS
BFLAT = N * H
B_PER_W = BFLAT // NW
C = 128
NCHUNK = B_PER_W // C

_LOSS_SCALE = (1.0 + BETA) / (N * D * H)


def _dist_kernel(e_ref, cb_ref, idx_ref, gidx_ref, loss_ref, cbn_ref):
    nb = pl.program_id(0)

    @pl.when(nb == 0)
    def _init():
        loss_ref[0, 0] = 0.0
        for h in range(H):
            cb = cb_ref[h]
            cbn_ref[h] = jnp.sum(cb * cb, axis=1)

    qs = []
    gqs = []
    block_loss = 0.0
    for h in range(H):
        z = e_ref[:, h, :]
        cb = cb_ref[h]
        zc = lax.dot_general(z, cb, (((1,), (1,)), ((), ())),
                             preferred_element_type=jnp.float32)
        rn = jnp.sum(z * z, axis=1, keepdims=True)
        d = (rn - 2.0 * zc) + cbn_ref[h][None, :]
        mn = jnp.min(d, axis=1, keepdims=True)
        iota = lax.broadcasted_iota(jnp.int32, d.shape, 1)
        q = jnp.min(jnp.where(d == mn, iota, K), axis=1)
        qs.append(q)
        gqs.append(q + h * K)
        block_loss = block_loss + jnp.sum(mn)

    idx_ref[0] = jnp.stack(qs, axis=1)
    gidx_ref[0] = jnp.stack(gqs, axis=1)
    loss_ref[0, 0] += block_loss

    @pl.when(nb == NB - 1)
    def _fin():
        loss_ref[0, 0] = loss_ref[0, 0] * _LOSS_SCALE


def _distances(embeds, codebooks):
    return pl.pallas_call(
        _dist_kernel,
        grid=(NB,),
        in_specs=[
            pl.BlockSpec((BN, H, D), lambda nb: (nb, 0, 0)),
            pl.BlockSpec((H, K, D), lambda nb: (0, 0, 0)),
        ],
        out_specs=[
            pl.BlockSpec((1, BN, H), lambda nb: (nb, 0, 0)),
            pl.BlockSpec((1, BN, H), lambda nb: (nb, 0, 0)),
            pl.BlockSpec(memory_space=pltpu.SMEM),
        ],
        out_shape=[
            jax.ShapeDtypeStruct((NB, BN, H), jnp.int32),
            jax.ShapeDtypeStruct((NB, BN, H), jnp.int32),
            jax.ShapeDtypeStruct((1, 1), jnp.float32),
        ],
        scratch_shapes=[pltpu.VMEM((H, K), jnp.float32)],
    )(embeds, codebooks)


@functools.lru_cache(maxsize=None)
def _make_gather():
    mesh = plsc.VectorSubcoreMesh(core_axis_name="c", subcore_axis_name="s")

    @functools.partial(
        pl.kernel,
        mesh=mesh,
        out_type=jax.ShapeDtypeStruct((BFLAT, D), jnp.float32),
        scratch_types=[
            pltpu.VMEM((NCHUNK, C), jnp.int32),
            pltpu.VMEM((2, C, D), jnp.float32),
            pltpu.SemaphoreType.DMA,
            pltpu.SemaphoreType.DMA,
        ],
    )
    def _gather(table_hbm, gidx_hbm, out_hbm, idx_v, rows_v, sem0, sem1):
        wid = lax.axis_index("s") * NC + lax.axis_index("c")
        base = wid * B_PER_W
        pltpu.sync_copy(gidx_hbm.at[wid], idx_v)
        sems = [sem0, sem1]
        pending = pltpu.async_copy(
            table_hbm.at[idx_v.at[0]], rows_v.at[0], sems[0])
        for c in range(NCHUNK):
            b = c % 2
            cur = pending
            if c + 1 < NCHUNK:
                nb = (c + 1) % 2
                pending = pltpu.async_copy(
                    table_hbm.at[idx_v.at[c + 1]], rows_v.at[nb], sems[nb])
            cur.wait()
            pltpu.sync_copy(rows_v.at[b], out_hbm.at[pl.ds(base + c * C, C)])

    return _gather


def kernel(embeds, codebooks):
    idx3, gidx3, loss = _distances(embeds, codebooks)
    indices = idx3.reshape(N, H)
    gflat = gidx3.reshape(NW, NCHUNK, C)
    table = codebooks.reshape(H * K, D)
    qflat = _make_gather()(table, gflat)
    quantized = qflat.reshape(N, H, D)
    return quantized, indices, loss[0, 0]


# speedup vs baseline: 3.1097x; 1.0221x over previous
import functools

import jax
import jax.numpy as jnp
from jax import lax
from jax.experimental import pallas as pl
from jax.experimental.pallas import tpu as pltpu
from jax.experimental.pallas import tpu_sc as plsc

H = 8
K = 1024
D = 256
N = 8192
BETA = 0.25

BN = 512
NB = N // BN

NC = 2
NS = 16
NW = NC * NS
BFLAT = N * H
B_PER_W = BFLAT // NW
C = 128
NCHUNK = B_PER_W // C

_LOSS_SCALE = (1.0 + BETA) / (N * D * H)


def _dist_kernel(e_ref, cb_ref, idx_ref, gidx_ref, loss_ref, cbn_ref):
    nb = pl.program_id(0)

    @pl.when(nb == 0)
    def _init():
        loss_ref[0, 0] = 0.0
        for h in range(H):
            cb = cb_ref[h]
            cbn_ref[h] = jnp.sum(cb * cb, axis=1)

    iota_f = lax.broadcasted_iota(jnp.int32, (BN, K), 1).astype(jnp.float32)
    qs = []
    gqs = []
    block_loss = 0.0
    for h in range(H):
        z = e_ref[:, h * D:(h + 1) * D]
        cb = cb_ref[h]
        zc = lax.dot_general(z, cb, (((1,), (1,)), ((), ())),
                             preferred_element_type=jnp.float32)
        rn = jnp.sum(z * z, axis=1, keepdims=True)
        d = (rn - 2.0 * zc) + cbn_ref[h][None, :]
        mn = jnp.min(d, axis=1, keepdims=True)
        qf = jnp.min(jnp.where(d == mn, iota_f, float(K)), axis=1)
        q = qf.astype(jnp.int32)
        qs.append(q)
        gqs.append(q + h * K)
        block_loss = block_loss + jnp.sum(mn)

    idx_ref[0] = jnp.stack(qs, axis=1)
    gidx_ref[0] = jnp.stack(gqs, axis=1)
    loss_ref[0, 0] += block_loss

    @pl.when(nb == NB - 1)
    def _fin():
        loss_ref[0, 0] = loss_ref[0, 0] * _LOSS_SCALE


def _distances(embeds, codebooks):
    return pl.pallas_call(
        _dist_kernel,
        grid=(NB,),
        in_specs=[
            pl.BlockSpec((BN, H * D), lambda nb: (nb, 0)),
            pl.BlockSpec((H, K, D), lambda nb: (0, 0, 0)),
        ],
        out_specs=[
            pl.BlockSpec((1, BN, H), lambda nb: (nb, 0, 0)),
            pl.BlockSpec((1, BN, H), lambda nb: (nb, 0, 0)),
            pl.BlockSpec(memory_space=pltpu.SMEM),
        ],
        out_shape=[
            jax.ShapeDtypeStruct((NB, BN, H), jnp.int32),
            jax.ShapeDtypeStruct((NB, BN, H), jnp.int32),
            jax.ShapeDtypeStruct((1, 1), jnp.float32),
        ],
        scratch_shapes=[pltpu.VMEM((H, K), jnp.float32)],
    )(embeds.reshape(N, H * D), codebooks)


@functools.lru_cache(maxsize=None)
def _make_gather():
    mesh = plsc.VectorSubcoreMesh(core_axis_name="c", subcore_axis_name="s")

    @functools.partial(
        pl.kernel,
        mesh=mesh,
        out_type=jax.ShapeDtypeStruct((BFLAT, D), jnp.float32),
        scratch_types=[
            pltpu.VMEM((NCHUNK, C), jnp.int32),
            pltpu.VMEM((2, C, D), jnp.float32),
            pltpu.SemaphoreType.DMA,
            pltpu.SemaphoreType.DMA,
        ],
    )
    def _gather(table_hbm, gidx_hbm, out_hbm, idx_v, rows_v, sem0, sem1):
        wid = lax.axis_index("s") * NC + lax.axis_index("c")
        base = wid * B_PER_W
        pltpu.sync_copy(gidx_hbm.at[wid], idx_v)
        sems = [sem0, sem1]
        pending = pltpu.async_copy(
            table_hbm.at[idx_v.at[0]], rows_v.at[0], sems[0])
        for c in range(NCHUNK):
            b = c % 2
            cur = pending
            if c + 1 < NCHUNK:
                nb = (c + 1) % 2
                pending = pltpu.async_copy(
                    table_hbm.at[idx_v.at[c + 1]], rows_v.at[nb], sems[nb])
            cur.wait()
            pltpu.sync_copy(rows_v.at[b], out_hbm.at[pl.ds(base + c * C, C)])

    return _gather


def kernel(embeds, codebooks):
    idx3, gidx3, loss = _distances(embeds, codebooks)
    indices = idx3.reshape(N, H)
    gflat = gidx3.reshape(NW, NCHUNK, C)
    table = codebooks.reshape(H * K, D)
    qflat = _make_gather()(table, gflat)
    quantized = qflat.reshape(N, H, D)
    return quantized, indices, loss[0, 0]
